# baseline (speedup 1.0000x reference)
import functools

import jax
import jax.numpy as jnp
from jax import lax
from jax.experimental import pallas as pl
from jax.experimental.pallas import tpu as pltpu
from jax.experimental.pallas import tpu_sc as plsc

N = 10000
E = 320000
C = 128
H = 8
DH = 16

NC = 2
NS = 16
NW = NC * NS
EPW = E // NW
CH = 80
NCHUNK = EPW // CH
SB = CH // 16
NPAD = 10240
RPT = NPAD // NS

_ROWB = 2000


def _pre_body(x_ref, g_ref, be_ref, wq_ref, bq_ref, wk_ref, bk_ref,
              wv_ref, bv_ref, q_ref, k_ref, v_ref):
    xb = x_ref[...]
    mu = jnp.mean(xb, axis=1, keepdims=True)
    xc = xb - mu
    var = jnp.mean(xc * xc, axis=1, keepdims=True)
    h = xc * lax.rsqrt(var + 1e-5) * g_ref[...] + be_ref[...]
    dn = (((1,), (1,)), ((), ()))
    q_ref[...] = lax.dot_general(h, wq_ref[...], dn,
                                 preferred_element_type=jnp.float32) + bq_ref[...]
    k_ref[...] = lax.dot_general(h, wk_ref[...], dn,
                                 preferred_element_type=jnp.float32) + bk_ref[...]
    v_ref[...] = lax.dot_general(h, wv_ref[...], dn,
                                 preferred_element_type=jnp.float32) + bv_ref[...]


def _tc_pre(x, g1, be1, Wq, bq, Wk, bk, Wv, bv):
    grid = (N // _ROWB,)
    row = pl.BlockSpec((_ROWB, C), lambda i: (i, 0))
    full = lambda shape: pl.BlockSpec(shape, lambda i: (0,) * len(shape))
    return pl.pallas_call(
        _pre_body,
        grid=grid,
        in_specs=[row, full((1, C)), full((1, C)),
                  full((C, C)), full((1, C)),
                  full((C, C)), full((1, C)),
                  full((C, C)), full((1, C))],
        out_specs=[row, row, row],
        out_shape=[jax.ShapeDtypeStruct((N, C), jnp.float32)] * 3,
    )(x, g1, be1, Wq, bq, Wk, bk, Wv, bv)


def _sc_body(q_hbm, k_hbm, v_hbm, si_hbm, di_hbm, out_hbm,
             qb, kb, vb, mb, sib, dib, zb, aggr, gsem):
    c = lax.axis_index("c")
    s = lax.axis_index("s")
    wid = c * NS + s
    base = wid * EPW
    iota = lax.iota(jnp.int32, 16)

    z16 = jnp.zeros((16,), jnp.float32)
    for r in range(16):
        for cb in range(C // 16):
            zb[r, pl.ds(cb * 16, 16)] = z16
    row0 = s * RPT

    def zero_body(j, carry):
        pltpu.sync_copy(zb, aggr.at[pl.ds(row0 + j * 16, 16)])
        return carry

    lax.fori_loop(0, RPT // 16, zero_body, 0)
    plsc.subcore_barrier()

    def chunk_body(i, carry):
        off = base + i * CH
        pltpu.sync_copy(si_hbm.at[pl.ds(off, CH)], sib)
        pltpu.sync_copy(di_hbm.at[pl.ds(off, CH)], dib)
        dq = pltpu.async_copy(q_hbm.at[dib], qb, gsem)
        dk = pltpu.async_copy(k_hbm.at[sib], kb, gsem)
        dv = pltpu.async_copy(v_hbm.at[sib], vb, gsem)
        dq.wait()
        dk.wait()
        dv.wait()

        def sub_body(b, carry2):
            rows = b * 16 + iota
            svec = []
            for h in range(H):
                acc = jnp.zeros((16,), jnp.float32)
                for d in range(DH):
                    colv = jnp.full((16,), h * DH + d, jnp.int32)
                    qc = plsc.load_gather(qb, [rows, colv])
                    kc = plsc.load_gather(kb, [rows, colv])
                    acc = acc + qc * kc
                svec.append(acc * 0.25)
            m = svec[0]
            for h in range(1, H):
                m = jnp.maximum(m, svec[h])
            evec = [jnp.exp(sv - m) for sv in svec]
            tot = evec[0]
            for h in range(1, H):
                tot = tot + evec[h]
            rinv = 1.0 / tot
            avec = [ev * rinv for ev in evec]
            for col in range(C):
                colv = jnp.full((16,), col, jnp.int32)
                vc = plsc.load_gather(vb, [rows, colv])
                plsc.store_scatter(mb, [rows, colv], vc * avec[col // DH])
            return carry2

        lax.fori_loop(0, SB, sub_body, 0)
        pltpu.sync_copy(mb, aggr.at[dib], add=True)
        return carry

    lax.fori_loop(0, NCHUNK, chunk_body, 0)

    plsc.subcore_barrier()
    pltpu.sync_copy(aggr.at[pl.ds(row0, RPT)],
                    out_hbm.at[c, pl.ds(row0, RPT), :])


_sc_edge = functools.partial(
    pl.kernel,
    out_type=jax.ShapeDtypeStruct((NC, NPAD, C), jnp.float32),
    mesh=plsc.VectorSubcoreMesh(core_axis_name="c", subcore_axis_name="s"),
    compiler_params=pltpu.CompilerParams(needs_layout_passes=False),
    scratch_types=[
        pltpu.VMEM((CH, C), jnp.float32),
        pltpu.VMEM((CH, C), jnp.float32),
        pltpu.VMEM((CH, C), jnp.float32),
        pltpu.VMEM((CH, C), jnp.float32),
        pltpu.VMEM((CH,), jnp.int32),
        pltpu.VMEM((CH,), jnp.int32),
        pltpu.VMEM((16, C), jnp.float32),
        pltpu.VMEM_SHARED((NPAD, C), jnp.float32),
        pltpu.SemaphoreType.DMA,
    ],
)(_sc_body)


def _post_body(p_ref, x_ref, wo_ref, bo_ref, g_ref, be_ref,
               w1_ref, b1_ref, w2_ref, b2_ref, o_ref):
    aggr = p_ref[0] + p_ref[1]
    dn = (((1,), (1,)), ((), ()))
    h2 = lax.dot_general(aggr, wo_ref[...], dn,
                         preferred_element_type=jnp.float32) + bo_ref[...] + x_ref[...]
    mu = jnp.mean(h2, axis=1, keepdims=True)
    xc = h2 - mu
    var = jnp.mean(xc * xc, axis=1, keepdims=True)
    f = xc * lax.rsqrt(var + 1e-5) * g_ref[...] + be_ref[...]
    f = jnp.maximum(lax.dot_general(f, w1_ref[...], dn,
                                    preferred_element_type=jnp.float32) + b1_ref[...], 0.0)
    f = lax.dot_general(f, w2_ref[...], dn,
                        preferred_element_type=jnp.float32) + b2_ref[...]
    o_ref[...] = f + h2


def _tc_post(part, x, Wo, bo, g2, be2, W1, bm1, W2, bm2):
    grid = (N // _ROWB,)
    row = pl.BlockSpec((_ROWB, C), lambda i: (i, 0))
    full = lambda shape: pl.BlockSpec(shape, lambda i: (0,) * len(shape))
    return pl.pallas_call(
        _post_body,
        grid=grid,
        in_specs=[pl.BlockSpec((NC, _ROWB, C), lambda i: (0, i, 0)), row,
                  full((C, C)), full((1, C)), full((1, C)), full((1, C)),
                  full((4 * C, C)), full((1, 4 * C)),
                  full((C, 4 * C)), full((1, C))],
        out_specs=row,
        out_shape=jax.ShapeDtypeStruct((N, C), jnp.float32),
    )(part, x, Wo, bo, g2, be2, W1, bm1, W2, bm2)


def kernel(x, edge_index, Wq, bq, Wk, bk, Wv, bv, Wo, bo,
           W1, bm1, W2, bm2, g1, be1, g2, be2):
    src = edge_index[0]
    dst = edge_index[1]
    r = lambda b: b.reshape(1, -1)
    q, k, v = _tc_pre(x, r(g1), r(be1), Wq, r(bq), Wk, r(bk), Wv, r(bv))
    part = _sc_edge(q, k, v, src, dst)
    return _tc_post(part, x, Wo, r(bo), r(g2), r(be2), W1, r(bm1), W2, r(bm2))
